# baseline (speedup 1.0000x reference)
import jax
import jax.numpy as jnp
from jax import lax
from jax.experimental import pallas as pl
from jax.experimental.pallas import tpu as pltpu
from jax.experimental.pallas import tpu_sc as plsc

D = 64
L = 3
C = 1024
NC = 2
NS = 16
NW = NC * NS
LANES = 16
KA = 1024
KB = 128
DP = 128


def _idx_body(ids_hbm, lookup_hbm, sidx_hbm, ids_v, cidx_v, sidx_v, sem):
    n_total = ids_hbm.shape[0]
    per_w = n_total // NW
    n_chunks = per_w // KA
    wid = lax.axis_index("s") * NC + lax.axis_index("c")

    def chunk_body(ci, carry):
        base = wid * per_w + ci * KA
        pltpu.sync_copy(ids_hbm.at[pl.ds(base, KA)], ids_v)
        for c in range(KA // LANES):
            v = ids_v[pl.ds(c * LANES, LANES)] * L
            for l in range(L):
                cidx_v[pl.ds(l * KA + c * LANES, LANES)] = v + l
        pltpu.async_copy(lookup_hbm.at[cidx_v], sidx_v, sem).wait()
        for l in range(L):
            off = jnp.int32(l * C + 1)
            for c in range(KA // LANES):
                s = pl.ds(l * KA + c * LANES, LANES)
                sidx_v[s] = sidx_v[s] + off
            pltpu.sync_copy(sidx_v.at[pl.ds(l * KA, KA)],
                            sidx_hbm.at[pl.ds(l * n_total + base, KA)])
        return carry

    lax.fori_loop(0, n_chunks, chunk_body, 0)


def _gather_body(ids_hbm, sidx_hbm, sid_hbm, ind_hbm, out_hbm,
                 ids_v, sidx_v, tmp_v, ind_v, out_v, sem_ind, sem_sid):
    n_total = ids_hbm.shape[0]
    per_w = n_total // NW
    n_chunks = per_w // KB
    wid = lax.axis_index("s") * NC + lax.axis_index("c")

    def chunk_body(ci, carry):
        base = wid * per_w + ci * KB
        pltpu.sync_copy(ids_hbm.at[pl.ds(base, KB)], ids_v)
        ind_dma = pltpu.async_copy(ind_hbm.at[ids_v], ind_v, sem_ind)
        for l in range(L):
            pltpu.sync_copy(sidx_hbm.at[pl.ds(l * n_total + base, KB)],
                            sidx_v.at[pl.ds(l * KB, KB)])
        pltpu.async_copy(sid_hbm.at[sidx_v], tmp_v, sem_sid).wait()
        ind_dma.wait()

        def add_body(i, carry2):
            for c in range(D // LANES):
                s = pl.ds(c * LANES, LANES)
                out_v[i, s] = (ind_v[i, s] + tmp_v[i, s]
                               + tmp_v[KB + i, s] + tmp_v[2 * KB + i, s])
            return carry2

        lax.fori_loop(0, KB, add_body, 0)
        pltpu.sync_copy(out_v, out_hbm.at[pl.ds(base, KB)])
        return carry

    lax.fori_loop(0, n_chunks, chunk_body, 0)


def _impl(ids, lookup_flat, sid_pad, ind_pad):
    n = ids.shape[0]
    mesh = plsc.VectorSubcoreMesh(core_axis_name="c", subcore_axis_name="s")
    sidx = pl.kernel(
        _idx_body,
        out_type=jax.ShapeDtypeStruct((L * n,), jnp.int32),
        mesh=mesh,
        compiler_params=pltpu.CompilerParams(use_tc_tiling_on_sc=False),
        scratch_types=[
            pltpu.VMEM((KA,), jnp.int32),
            pltpu.VMEM((L * KA,), jnp.int32),
            pltpu.VMEM((L * KA,), jnp.int32),
            pltpu.SemaphoreType.DMA,
        ],
    )(ids, lookup_flat)

    out = pl.kernel(
        _gather_body,
        out_type=jax.ShapeDtypeStruct((n, DP), jnp.float32),
        mesh=mesh,
        compiler_params=pltpu.CompilerParams(use_tc_tiling_on_sc=True),
        scratch_types=[
            pltpu.VMEM((KB,), jnp.int32),
            pltpu.VMEM((L * KB,), jnp.int32),
            pltpu.VMEM((L * KB, DP), jnp.float32),
            pltpu.VMEM((KB, DP), jnp.float32),
            pltpu.VMEM((KB, DP), jnp.float32),
            pltpu.SemaphoreType.DMA,
            pltpu.SemaphoreType.DMA,
        ],
    )(ids, sidx, sid_pad, ind_pad)
    return out


_PAD_BR = 4096


def _pad_tc_body(x_ref, o_ref):
    o_ref[:, :D] = x_ref[...]


def _pad_tc(table):
    nrow = table.shape[0]
    grid = (nrow + _PAD_BR - 1) // _PAD_BR
    return pl.pallas_call(
        _pad_tc_body,
        grid=(grid,),
        in_specs=[pl.BlockSpec((_PAD_BR, D), lambda i: (i, 0))],
        out_specs=pl.BlockSpec((_PAD_BR, DP), lambda i: (i, 0)),
        out_shape=jax.ShapeDtypeStruct((nrow, DP), jnp.float32),
    )(table)


def kernel(item_ids, lookup, codebook, sid_table, ind_table):
    b, t = item_ids.shape
    ids = item_ids.reshape(-1)
    lookup_flat = lookup.reshape(-1)
    sid_pad = _pad_tc(sid_table)
    ind_pad = _pad_tc(ind_table)
    out = _impl(ids, lookup_flat, sid_pad, ind_pad)
    return out[:, :D].reshape(b, t, D)


# speedup vs baseline: 2.8761x; 2.8761x over previous
import jax
import jax.numpy as jnp
from jax import lax
from jax.experimental import pallas as pl
from jax.experimental.pallas import tpu as pltpu
from jax.experimental.pallas import tpu_sc as plsc

D = 64
L = 3
C = 1024
NC = 2
NS = 16
NW = NC * NS
LANES = 16
K = 256


def _sc_body(ids_hbm, lk0_hbm, lk1_hbm, lk2_hbm, sid_hbm, ind_hbm, out_hbm,
             ids_v, sidx_v, tmp_v, acc_v, sem_codes, sem_ind, sem_sid):
    n_total = ids_hbm.shape[0]
    per_w = n_total // NW
    n_chunks = per_w // K
    wid = lax.axis_index("s") * NC + lax.axis_index("c")

    def chunk_body(ci, carry):
        base = wid * per_w + ci * K
        pltpu.sync_copy(ids_hbm.at[pl.ds(base, K)], ids_v)
        ind_dma = pltpu.async_copy(ind_hbm.at[ids_v], acc_v, sem_ind)
        dmas = [pltpu.async_copy(lk_hbm.at[ids_v],
                                 sidx_v.at[pl.ds(l * K, K)], sem_codes)
                for l, lk_hbm in enumerate((lk0_hbm, lk1_hbm, lk2_hbm))]
        for dma in dmas:
            dma.wait()
        for l in range(L):
            off = jnp.int32(l * C + 1)
            for c in range(K // LANES):
                s = pl.ds(l * K + c * LANES, LANES)
                sidx_v[s] = sidx_v[s] + off
        pltpu.async_copy(sid_hbm.at[sidx_v], tmp_v, sem_sid).wait()
        ind_dma.wait()

        def add_body(i, carry2):
            for c in range(D // LANES):
                s = pl.ds(c * LANES, LANES)
                acc_v[i, s] = (acc_v[i, s] + tmp_v[i, s]
                               + tmp_v[K + i, s] + tmp_v[2 * K + i, s])
            return carry2

        lax.fori_loop(0, K, add_body, 0)
        pltpu.sync_copy(acc_v, out_hbm.at[pl.ds(base, K)])
        return carry

    lax.fori_loop(0, n_chunks, chunk_body, 0)


def _impl(ids, lk0, lk1, lk2, sid_table, ind_table):
    n = ids.shape[0]
    mesh = plsc.VectorSubcoreMesh(core_axis_name="c", subcore_axis_name="s")
    fn = pl.kernel(
        _sc_body,
        out_type=jax.ShapeDtypeStruct((n, D), jnp.float32),
        mesh=mesh,
        compiler_params=pltpu.CompilerParams(use_tc_tiling_on_sc=False),
        scratch_types=[
            pltpu.VMEM((K,), jnp.int32),
            pltpu.VMEM((L * K,), jnp.int32),
            pltpu.VMEM((L * K, D), jnp.float32),
            pltpu.VMEM((K, D), jnp.float32),
            pltpu.SemaphoreType.DMA,
            pltpu.SemaphoreType.DMA,
            pltpu.SemaphoreType.DMA,
        ],
    )
    return fn(ids, lk0, lk1, lk2, sid_table, ind_table)


def kernel(item_ids, lookup, codebook, sid_table, ind_table):
    b, t = item_ids.shape
    ids = item_ids.reshape(-1)
    lk0, lk1, lk2 = (lookup[:, l] for l in range(L))
    out = _impl(ids, lk0, lk1, lk2, sid_table, ind_table)
    return out.reshape(b, t, D)


# speedup vs baseline: 3.1808x; 1.1059x over previous
import jax
import jax.numpy as jnp
from jax import lax
from jax.experimental import pallas as pl
from jax.experimental.pallas import tpu as pltpu
from jax.experimental.pallas import tpu_sc as plsc

D = 64
L = 3
C = 1024
NC = 2
NS = 16
NW = NC * NS
LANES = 16
K = 160


def _sc_body(ids_hbm, lk0_hbm, lk1_hbm, lk2_hbm, sid_hbm, ind_hbm, out_hbm,
             ids_v0, sidx_v0, tmp_v0, acc_v0, ids_v1, sidx_v1, tmp_v1, acc_v1,
             semc0, semi0, sems0, semc1, semi1, sems1):
    n_total = ids_hbm.shape[0]
    per_w = n_total // NW
    n_chunks = per_w // K
    wid = lax.axis_index("s") * NC + lax.axis_index("c")
    lks = (lk0_hbm, lk1_hbm, lk2_hbm)
    bufs = ((ids_v0, sidx_v0, tmp_v0, acc_v0, semc0, semi0, sems0),
            (ids_v1, sidx_v1, tmp_v1, acc_v1, semc1, semi1, sems1))

    def s1(ci, buf):
        ids_v, sidx_v, tmp_v, acc_v, semc, semi, sems = buf
        base = wid * per_w + ci * K
        pltpu.sync_copy(ids_hbm.at[pl.ds(base, K)], ids_v)
        pltpu.async_copy(ind_hbm.at[ids_v], acc_v, semi)
        for l in range(L):
            pltpu.async_copy(lks[l].at[ids_v], sidx_v.at[pl.ds(l * K, K)],
                             semc)

    def s2(buf):
        ids_v, sidx_v, tmp_v, acc_v, semc, semi, sems = buf
        for l in range(L):
            pltpu.make_async_copy(lks[l].at[ids_v],
                                  sidx_v.at[pl.ds(l * K, K)], semc).wait()
        for l in range(L):
            off = jnp.int32(l * C + 1)
            for c in range(K // LANES):
                s = pl.ds(l * K + c * LANES, LANES)
                sidx_v[s] = sidx_v[s] + off
        pltpu.async_copy(sid_hbm.at[sidx_v], tmp_v, sems)

    def s3(ci, buf):
        ids_v, sidx_v, tmp_v, acc_v, semc, semi, sems = buf
        base = wid * per_w + ci * K
        pltpu.make_async_copy(sid_hbm.at[sidx_v], tmp_v, sems).wait()
        pltpu.make_async_copy(ind_hbm.at[ids_v], acc_v, semi).wait()

        def add_body(i, carry2):
            for c in range(D // LANES):
                s = pl.ds(c * LANES, LANES)
                acc_v[i, s] = (acc_v[i, s] + tmp_v[i, s]
                               + tmp_v[K + i, s] + tmp_v[2 * K + i, s])
            return carry2

        lax.fori_loop(0, K, add_body, 0)
        pltpu.sync_copy(acc_v, out_hbm.at[pl.ds(base, K)])

    s1(0, bufs[0])
    s2(bufs[0])

    def pair_body(p, carry):
        for par in range(2):
            ci = 2 * p + par
            nxt_buf = bufs[1 - par]

            @pl.when(ci + 1 < n_chunks)
            def _():
                s1(ci + 1, nxt_buf)
                s2(nxt_buf)

            s3(ci, bufs[par])
        return carry

    lax.fori_loop(0, n_chunks // 2, pair_body, 0)


def _impl(ids, lk0, lk1, lk2, sid_table, ind_table):
    n = ids.shape[0]
    mesh = plsc.VectorSubcoreMesh(core_axis_name="c", subcore_axis_name="s")
    buf_set = [
        pltpu.VMEM((K,), jnp.int32),
        pltpu.VMEM((L * K,), jnp.int32),
        pltpu.VMEM((L * K, D), jnp.float32),
        pltpu.VMEM((K, D), jnp.float32),
    ]
    fn = pl.kernel(
        _sc_body,
        out_type=jax.ShapeDtypeStruct((n, D), jnp.float32),
        mesh=mesh,
        compiler_params=pltpu.CompilerParams(use_tc_tiling_on_sc=False),
        scratch_types=buf_set + buf_set + [pltpu.SemaphoreType.DMA] * 6,
    )
    return fn(ids, lk0, lk1, lk2, sid_table, ind_table)


def kernel(item_ids, lookup, codebook, sid_table, ind_table):
    b, t = item_ids.shape
    ids = item_ids.reshape(-1)
    lk0, lk1, lk2 = (lookup[:, l] for l in range(L))
    out = _impl(ids, lk0, lk1, lk2, sid_table, ind_table)
    return out.reshape(b, t, D)
